# Initial kernel scaffold; baseline (speedup 1.0000x reference)
import jax
import jax.numpy as jnp
from jax.experimental import pallas as pl


def kernel(x, edge_index, W1, b1, W2, b2, W3, b3, bn1_g, bn1_b, bn2_g, bn2_b, bn3_g, bn3_b, Wg, att_src, att_dst, bg):
    raise NotImplementedError("write your pallas kernel here")


# speedup vs baseline: 33.0261x; 33.0261x over previous
import functools

import jax
import jax.numpy as jnp
from jax import lax
from jax.experimental import pallas as pl
from jax.experimental.pallas import tpu as pltpu
from jax.experimental.pallas import tpu_sc as plsc

N = 10000
E = 320000
D_IN = 128
H = 64
DP = 128

NC = 2
NS = 16
NW = NC * NS
EPT = E // NW
BLK = 80
BLKS = EPT // BLK
RPT0 = 624
TAILO = (NS - 1) * RPT0
TAILN = N - TAILO

BLK2 = 64
BLKS2 = 157
NH = N // 2
RPT0B = 312
TAILOB = (NS - 1) * RPT0B
TAILNB = NH - TAILOB

RB = 1000
GB = N // RB

_f32 = jnp.float32


def _sc_mesh():
    return plsc.VectorSubcoreMesh(core_axis_name="c", subcore_axis_name="s")


def _striped_copy(src, dst, s):
    pltpu.sync_copy(src.at[pl.ds(s * RPT0, RPT0)], dst.at[pl.ds(s * RPT0, RPT0)])

    @pl.when(s == NS - 1)
    def _():
        pltpu.sync_copy(src.at[pl.ds(TAILO, TAILN)], dst.at[pl.ds(TAILO, TAILN)])


def _striped_copy_half(src, dst, s):
    pltpu.sync_copy(src.at[pl.ds(s * RPT0B, RPT0B)],
                    dst.at[pl.ds(s * RPT0B, RPT0B)])

    @pl.when(s == NS - 1)
    def _():
        pltpu.sync_copy(src.at[pl.ds(TAILOB, TAILNB)],
                        dst.at[pl.ds(TAILOB, TAILNB)])


@jax.jit
def _sc_segment_sum(src3, dst3, g, zeros_nd):
    @functools.partial(
        pl.kernel,
        out_type=jax.ShapeDtypeStruct((NC, N, DP), _f32),
        mesh=_sc_mesh(),
        scratch_types=[
            pltpu.VMEM((BLKS, BLK), jnp.int32),
            pltpu.VMEM((BLKS, BLK), jnp.int32),
            pltpu.VMEM((BLK, DP), _f32),
            pltpu.VMEM_SHARED((N, DP), _f32),
            pltpu.SemaphoreType.DMA,
        ],
    )
    def k(src_hbm, dst_hbm, g_hbm, z_hbm, out_hbm, src_t, dst_t, rows, acc, sem):
        c = lax.axis_index("c")
        s = lax.axis_index("s")
        wid = c * NS + s
        pltpu.sync_copy(src_hbm.at[wid], src_t)
        pltpu.sync_copy(dst_hbm.at[wid], dst_t)
        _striped_copy(z_hbm, acc, s)
        plsc.subcore_barrier()

        def body(b, carry):
            pltpu.async_copy(g_hbm.at[src_t.at[b]], rows, sem).wait()
            pltpu.sync_copy(rows, acc.at[dst_t.at[b]], add=True)
            return carry

        lax.fori_loop(0, BLKS, body, 0)
        plsc.subcore_barrier()
        _striped_copy(acc, out_hbm.at[c], s)

    return k(src3, dst3, g, zeros_nd)


@jax.jit
def _sc_degree(dst3, ones_blk, zeros_n16):
    @functools.partial(
        pl.kernel,
        out_type=jax.ShapeDtypeStruct((NC, N, DP), _f32),
        mesh=_sc_mesh(),
        scratch_types=[
            pltpu.VMEM((BLKS, BLK), jnp.int32),
            pltpu.VMEM((BLK, DP), _f32),
            pltpu.VMEM_SHARED((N, DP), _f32),
        ],
    )
    def k(dst_hbm, ones_hbm, z_hbm, out_hbm, dst_t, ones_t, acc):
        c = lax.axis_index("c")
        s = lax.axis_index("s")
        wid = c * NS + s
        pltpu.sync_copy(dst_hbm.at[wid], dst_t)
        pltpu.sync_copy(ones_hbm, ones_t)
        _striped_copy(z_hbm, acc, s)
        plsc.subcore_barrier()

        def body(b, carry):
            pltpu.sync_copy(ones_t, acc.at[dst_t.at[b]], add=True)
            return carry

        lax.fori_loop(0, BLKS, body, 0)
        plsc.subcore_barrier()
        _striped_copy(acc, out_hbm.at[c], s)

    return k(dst3, ones_blk, zeros_n16)


@jax.jit
def _sc_gat(comb3, dstp3, hgxp, zeros_nh):
    @functools.partial(
        pl.kernel,
        out_type=jax.ShapeDtypeStruct((NC, NH, DP), _f32),
        mesh=_sc_mesh(),
        scratch_types=[
            pltpu.VMEM((BLKS2, 2 * BLK2), jnp.int32),
            pltpu.VMEM((BLKS2, BLK2), jnp.int32),
            pltpu.VMEM((2 * BLK2, DP), _f32),
            pltpu.VMEM((BLK2, DP), _f32),
            pltpu.VMEM((8, DP), _f32),
            pltpu.VMEM_SHARED((NH + 8, DP), _f32),
            pltpu.SemaphoreType.DMA,
        ],
    )
    def k(comb_hbm, dst_hbm, hgx_hbm, z_hbm, o_out,
          comb_t, dst_t, rows, rows_w, mrow, acc, sem):
        c = lax.axis_index("c")
        s = lax.axis_index("s")
        wid = c * NS + s
        pltpu.sync_copy(comb_hbm.at[wid], comb_t)
        pltpu.sync_copy(dst_hbm.at[wid], dst_t)
        pltpu.sync_copy(hgx_hbm.at[pl.ds(N, 8)], mrow)
        _striped_copy_half(z_hbm, acc, s)
        zero16 = jnp.zeros((16,), _f32)

        def zbody(j, carry):
            for q in range(3):
                rows_w[j, pl.ds(80 + q * 16, 16)] = zero16
            return carry

        lax.fori_loop(0, BLK2, zbody, 0)
        plsc.subcore_barrier()
        mbv = mrow[1, pl.ds(H, 16)]

        def body(b, carry):
            pltpu.async_copy(hgx_hbm.at[comb_t.at[b]], rows, sem).wait()

            def wbody(j, carry2):
                av = rows[j, pl.ds(H, 16)]
                dv = rows[BLK2 + j, pl.ds(H + 16, 16)]
                z = av + dv
                e = jnp.maximum(z, 0.2 * z)
                t = jnp.exp(e - mbv)
                rows_w[j, pl.ds(H, 16)] = t
                for h in range(4):
                    rows_w[j, pl.ds(h * 16, 16)] = (
                        rows[j, pl.ds(h * 16, 16)] * t[h])
                return carry2

            lax.fori_loop(0, BLK2, wbody, 0)
            pltpu.sync_copy(rows_w, acc.at[dst_t.at[b]], add=True)
            return carry

        lax.fori_loop(0, BLKS2, body, 0)
        plsc.subcore_barrier()
        _striped_copy_half(acc, o_out.at[c], s)

    return k(comb3, dstp3, hgxp, zeros_nh)


def _t1_body(degp_ref, x_ref, w_ref, g_ref, dis_ref):
    deg = degp_ref[0, :, 0:1] + degp_ref[1, :, 0:1] + 1.0
    dis = lax.rsqrt(deg)
    hw = jnp.dot(x_ref[...], w_ref[...], preferred_element_type=_f32)
    g_ref[...] = hw * dis
    dis_ref[...] = dis


def _tc_first(degp, x, w1p):
    return pl.pallas_call(
        _t1_body,
        grid=(GB,),
        in_specs=[
            pl.BlockSpec((NC, RB, DP), lambda i: (0, i, 0)),
            pl.BlockSpec((RB, D_IN), lambda i: (i, 0)),
            pl.BlockSpec((D_IN, DP), lambda i: (0, 0)),
        ],
        out_specs=[
            pl.BlockSpec((RB, DP), lambda i: (i, 0)),
            pl.BlockSpec((RB, 1), lambda i: (i, 0)),
        ],
        out_shape=[
            jax.ShapeDtypeStruct((N, DP), _f32),
            jax.ShapeDtypeStruct((N, 1), _f32),
        ],
    )(degp, x, w1p)


def _stats_body(acc_ref, g_ref, dis_ref, b_ref, z_ref, st_ref, st_acc):
    i = pl.program_id(0)

    @pl.when(i == 0)
    def _():
        st_acc[...] = jnp.zeros_like(st_acc)

    z = dis_ref[...] * (acc_ref[0] + acc_ref[1] + g_ref[...]) + b_ref[...]
    z_ref[...] = z
    st_acc[0:1, :] += jnp.sum(z, axis=0, keepdims=True)
    st_acc[1:2, :] += jnp.sum(z * z, axis=0, keepdims=True)

    @pl.when(i == GB - 1)
    def _():
        st_ref[...] = st_acc[...]


def _tc_conv_stats(acc, g, dis, b):
    return pl.pallas_call(
        _stats_body,
        grid=(GB,),
        in_specs=[
            pl.BlockSpec((NC, RB, DP), lambda i: (0, i, 0)),
            pl.BlockSpec((RB, DP), lambda i: (i, 0)),
            pl.BlockSpec((RB, 1), lambda i: (i, 0)),
            pl.BlockSpec((1, DP), lambda i: (0, 0)),
        ],
        out_specs=[
            pl.BlockSpec((RB, DP), lambda i: (i, 0)),
            pl.BlockSpec((2, DP), lambda i: (0, 0)),
        ],
        out_shape=[
            jax.ShapeDtypeStruct((N, DP), _f32),
            jax.ShapeDtypeStruct((2, DP), _f32),
        ],
        scratch_shapes=[pltpu.VMEM((2, DP), _f32)],
    )(acc, g, dis, b)


def _apply_body(z_ref, st_ref, gam_ref, bet_ref, dis_ref, w_ref, g_ref):
    inv_n = 1.0 / N
    mu = st_ref[0:1, :] * inv_n
    var = st_ref[1:2, :] * inv_n - mu * mu
    y = (z_ref[...] - mu) * lax.rsqrt(var + 1e-5) * gam_ref[...] + bet_ref[...]
    h = jnp.maximum(y, 0.0)
    g_ref[...] = jnp.dot(h, w_ref[...], preferred_element_type=_f32) * dis_ref[...]


def _tc_bn_relu_mm(z, st, gam, bet, dis, wp):
    return pl.pallas_call(
        _apply_body,
        grid=(GB,),
        in_specs=[
            pl.BlockSpec((RB, DP), lambda i: (i, 0)),
            pl.BlockSpec((2, DP), lambda i: (0, 0)),
            pl.BlockSpec((1, DP), lambda i: (0, 0)),
            pl.BlockSpec((1, DP), lambda i: (0, 0)),
            pl.BlockSpec((RB, 1), lambda i: (i, 0)),
            pl.BlockSpec((DP, DP), lambda i: (0, 0)),
        ],
        out_specs=pl.BlockSpec((RB, DP), lambda i: (i, 0)),
        out_shape=jax.ShapeDtypeStruct((N, DP), _f32),
    )(z, st, gam, bet, dis, wp)


def _gatprep_body(z_ref, st_ref, gam_ref, bet_ref, wg_ref, asm_ref, adm_ref,
                  hg_ref, mb_ref, m_acc):
    i = pl.program_id(0)
    inv_n = 1.0 / N
    mu = st_ref[0:1, :] * inv_n
    var = st_ref[1:2, :] * inv_n - mu * mu
    y = (z_ref[...] - mu) * lax.rsqrt(var + 1e-5) * gam_ref[...] + bet_ref[...]
    h3 = jnp.maximum(y, 0.0)
    hg = jnp.dot(h3, wg_ref[...], preferred_element_type=_f32)
    a_s = jnp.dot(hg, asm_ref[...], preferred_element_type=_f32)
    a_d = jnp.dot(hg, adm_ref[...], preferred_element_type=_f32)
    nb = a_s.shape[0]
    hg_ref[...] = jnp.concatenate(
        [hg[:, 0:H], a_s, jnp.zeros((nb, 12), _f32),
         a_d, jnp.zeros((nb, DP - H - 20), _f32)], axis=1)
    bmax_s = jnp.max(a_s, axis=0, keepdims=True)
    bmax_d = jnp.max(a_d, axis=0, keepdims=True)

    @pl.when(i == 0)
    def _():
        m_acc[...] = jnp.full_like(m_acc, -3e38)

    m_acc[0:1, :] = jnp.maximum(m_acc[0:1, :], bmax_s)
    m_acc[1:2, :] = jnp.maximum(m_acc[1:2, :], bmax_d)

    @pl.when(i == GB - 1)
    def _():
        zz = m_acc[0:1, :] + m_acc[1:2, :]
        mb4 = jnp.maximum(zz, 0.2 * zz)
        mb_ref[...] = jnp.concatenate(
            [mb4, jnp.full((1, 12), 3e38, _f32)], axis=1)


def _tc_gat_prep(z, st, gam, bet, wgp, asm, adm):
    return pl.pallas_call(
        _gatprep_body,
        grid=(GB,),
        in_specs=[
            pl.BlockSpec((RB, DP), lambda i: (i, 0)),
            pl.BlockSpec((2, DP), lambda i: (0, 0)),
            pl.BlockSpec((1, DP), lambda i: (0, 0)),
            pl.BlockSpec((1, DP), lambda i: (0, 0)),
            pl.BlockSpec((DP, DP), lambda i: (0, 0)),
            pl.BlockSpec((DP, 4), lambda i: (0, 0)),
            pl.BlockSpec((DP, 4), lambda i: (0, 0)),
        ],
        out_specs=[
            pl.BlockSpec((RB, DP), lambda i: (i, 0)),
            pl.BlockSpec((1, 16), lambda i: (0, 0)),
        ],
        out_shape=[
            jax.ShapeDtypeStruct((N, DP), _f32),
            jax.ShapeDtypeStruct((1, 16), _f32),
        ],
        scratch_shapes=[pltpu.VMEM((2, 4), _f32)],
    )(z, st, gam, bet, wgp, asm, adm)


def _final_body(op_ref, hgx_ref, mb_ref, bg_ref, o_ref):
    a_s = hgx_ref[:, H:H + 4]
    a_d = hgx_ref[:, H + 16:H + 20]
    zz = a_s + a_d
    ee = jnp.maximum(zz, 0.2 * zz)
    ts = jnp.exp(ee - mb_ref[0:1, 0:4])
    s_tot = op_ref[0, :, H:H + 4] + op_ref[1, :, H:H + 4] + ts
    outs = []
    for h in range(4):
        num_h = (op_ref[0, :, h * 16:(h + 1) * 16]
                 + op_ref[1, :, h * 16:(h + 1) * 16]
                 + ts[:, h:h + 1] * hgx_ref[:, h * 16:(h + 1) * 16])
        outs.append(num_h / (s_tot[:, h:h + 1] + 1e-16))
    o = (outs[0] + outs[1] + outs[2] + outs[3]) * 0.25 + bg_ref[...]
    o = jnp.maximum(o, 0.0)
    m = jnp.max(o, axis=1, keepdims=True)
    zc = o - m
    o_ref[...] = zc - jnp.log(jnp.sum(jnp.exp(zc), axis=1, keepdims=True))


def _tc_final(op, hgx, mb, bg):
    return pl.pallas_call(
        _final_body,
        grid=(GB,),
        in_specs=[
            pl.BlockSpec((NC, RB, DP), lambda i: (0, i, 0)),
            pl.BlockSpec((RB, DP), lambda i: (i, 0)),
            pl.BlockSpec((1, 16), lambda i: (0, 0)),
            pl.BlockSpec((1, 16), lambda i: (0, 0)),
        ],
        out_specs=pl.BlockSpec((RB, 16), lambda i: (i, 0)),
        out_shape=jax.ShapeDtypeStruct((N, 16), _f32),
    )(op, hgx, mb, bg)


def _pad2(w, r, c):
    return jnp.zeros((r, c), _f32).at[:w.shape[0], :w.shape[1]].set(w)


def _pad_row(v):
    return jnp.zeros((1, DP), _f32).at[0, :v.shape[0]].set(v)


def kernel(x, edge_index, W1, b1, W2, b2, W3, b3, bn1_g, bn1_b, bn2_g, bn2_b,
           bn3_g, bn3_b, Wg, att_src, att_dst, bg):
    src3 = edge_index[0].reshape(NW, BLKS, BLK)
    dst3 = edge_index[1].reshape(NW, BLKS, BLK)

    zeros_nd = jnp.zeros((N, DP), _f32)
    ones_blk = jnp.ones((BLK, DP), _f32)

    w1p = _pad2(W1, D_IN, DP)
    w2p = _pad2(W2, DP, DP)
    w3p = _pad2(W3, DP, DP)
    wgp = _pad2(Wg, DP, DP)

    degp = _sc_degree(dst3, ones_blk, zeros_nd)

    g1, dis = _tc_first(degp, x, w1p)
    acc1 = _sc_segment_sum(src3, dst3, g1, zeros_nd)
    z1, st1 = _tc_conv_stats(acc1, g1, dis, _pad_row(b1))

    g2 = _tc_bn_relu_mm(z1, st1, _pad_row(bn1_g), _pad_row(bn1_b), dis, w2p)
    acc2 = _sc_segment_sum(src3, dst3, g2, zeros_nd)
    z2, st2 = _tc_conv_stats(acc2, g2, dis, _pad_row(b2))

    g3 = _tc_bn_relu_mm(z2, st2, _pad_row(bn2_g), _pad_row(bn2_b), dis, w3p)
    acc3 = _sc_segment_sum(src3, dst3, g3, zeros_nd)
    z3, st3 = _tc_conv_stats(acc3, g3, dis, _pad_row(b3))

    eye4 = jnp.eye(4, dtype=_f32)
    asm = _pad2((att_src[:, :, None] * eye4[:, None, :]).reshape(H, 4), DP, 4)
    adm = _pad2((att_dst[:, :, None] * eye4[:, None, :]).reshape(H, 4), DP, 4)
    hgx, mb = _tc_gat_prep(z3, st3, _pad_row(bn3_g), _pad_row(bn3_b),
                           wgp, asm, adm)

    pad = BLKS2 * BLK2 * NW - E
    idt = edge_index.dtype
    srcp = jnp.concatenate([edge_index[0], jnp.full((pad,), N, idt)])
    dstp = jnp.concatenate([edge_index[1], jnp.zeros((pad,), idt)])
    src2 = srcp.reshape(NW, BLKS2, BLK2)
    dst2 = dstp.reshape(NW, BLKS2, BLK2)
    comb3 = jnp.concatenate([src2, dst2], axis=2)
    extra = (jnp.zeros((8, DP), _f32)
             .at[0, H:H + 4].set(-1e38)
             .at[1, H:H + 16].set(mb[0]))
    hgxp = jnp.concatenate([hgx, extra], axis=0)
    zeros_nh = jnp.zeros((NH, DP), _f32)
    dstA = jnp.where(dst2 < NH, dst2, NH)
    dstB = jnp.where(dst2 >= NH, dst2 - NH, NH)
    opA = _sc_gat(comb3, dstA, hgxp, zeros_nh)
    opB = _sc_gat(comb3, dstB, hgxp, zeros_nh)
    op = jnp.concatenate([opA, opB], axis=1)

    return _tc_final(op, hgx, mb, bg.reshape(1, 16))
